# baseline (speedup 1.0000x reference)
import functools

import jax
import jax.numpy as jnp
from jax import lax
from jax.experimental import pallas as pl
from jax.experimental.pallas import tpu as pltpu
from jax.experimental.pallas import tpu_sc as plsc

N = 320000
D = 128
MAXP = 10
NC, NS = 2, 16
NW = NC * NS
CH = 400
SPLIT_M = 11
NSC = 12800 * SPLIT_M
NT = N - NSC
RPW = NSC // NW
NCH = RPW // CH
VPC = CH // 16
KD = D // 16
BT = 8192
NBT = (NT + BT - 1) // BT
GB = BT // 32
NEG = -1e30


def _sc_body(feats_hbm, ids_hbm, part_out, cnt_out,
             ids_v, buf, acc_s, acc_m, cnt_v, sem_a, sem_b):
    cid = lax.axis_index("c")
    sid = lax.axis_index("s")
    wid = cid * NS + sid
    base = NT + wid * RPW
    CHD = CH * D

    pltpu.sync_copy(ids_hbm.at[pl.ds(base, RPW)], ids_v)

    zero = jnp.zeros((16,), jnp.float32)
    neg = jnp.full((16,), NEG, jnp.float32)

    def init_body(r, c):
        acc_s[pl.ds(r * 16, 16)] = zero
        acc_m[pl.ds(r * 16, 16)] = neg
        return c
    lax.fori_loop(0, MAXP * KD, init_body, 0)
    cnt_v[...] = zero
    lanes = lax.iota(jnp.int32, 16)

    def start_copy(g, half, sem):
        pltpu.async_copy(feats_hbm.at[pl.ds((base + g * CH) * D, CHD)],
                         buf.at[pl.ds(half * CHD, CHD)], sem)

    def wait_copy(half, sem):
        pltpu.make_async_copy(feats_hbm.at[pl.ds(0, CHD)],
                              buf.at[pl.ds(half * CHD, CHD)], sem).wait()

    def process(g, half):
        row0 = g * CH
        boff = half * CHD
        id0 = ids_v[pl.ds(row0, 16)][0]
        idl = ids_v[pl.ds(row0 + CH - 16, 16)][15]

        def los_uniform():
            return tuple(jnp.where(id0 >= p, jnp.int32(0), jnp.int32(CH))
                         for p in range(1, MAXP))

        def los_general():
            def cnt_body(v, carry):
                idv = ids_v[pl.ds(row0 + v * 16, 16)]
                return tuple(carry[p - 1]
                             + plsc.all_reduce_population_count(idv < p)
                             for p in range(1, MAXP))
            cvecs = lax.fori_loop(
                0, VPC, cnt_body,
                tuple(jnp.zeros((16,), jnp.int32) for _ in range(MAXP - 1)))
            return tuple(cv[0] for cv in cvecs)

        mids = lax.cond(id0 == idl, los_uniform, los_general)
        los = [jnp.int32(0)] + list(mids) + [jnp.int32(CH)]

        for p in range(MAXP):
            lo, hi = los[p], los[p + 1]

            @pl.when(hi > lo)
            def _():
                def row_body(j, carry):
                    ss, mm = carry
                    off = boff + j * D
                    ss2, mm2 = [], []
                    for k in range(KD):
                        v = buf[pl.ds(off + 16 * k, 16)]
                        ss2.append(ss[k] + v)
                        mm2.append(jnp.maximum(mm[k], v))
                    return tuple(ss2), tuple(mm2)

                ss, mm = lax.fori_loop(
                    lo, hi, row_body,
                    (tuple(zero for _ in range(KD)),
                     tuple(neg for _ in range(KD))))
                for k in range(KD):
                    sl = pl.ds(p * D + 16 * k, 16)
                    acc_s[sl] = acc_s[sl] + ss[k]
                    acc_m[sl] = jnp.maximum(acc_m[sl], mm[k])
                cnt_v[...] = cnt_v[...] + jnp.where(
                    lanes == p, (hi - lo).astype(jnp.float32),
                    jnp.float32(0.0))

    start_copy(0, 0, sem_a)

    def pair_body(gp, c):
        g0 = gp * 2
        start_copy(g0 + 1, 1, sem_b)
        wait_copy(0, sem_a)
        process(g0, 0)
        start_copy(g0 + 2, 0, sem_a)
        wait_copy(1, sem_b)
        process(g0 + 1, 1)
        return c

    lax.fori_loop(0, (NCH - 1) // 2, pair_body, 0)
    wait_copy(0, sem_a)
    process(NCH - 1, 0)

    pltpu.sync_copy(acc_s, part_out.at[wid, pl.ds(0, MAXP * D)])
    pltpu.sync_copy(acc_m, part_out.at[wid, pl.ds(MAXP * D, MAXP * D)])
    pltpu.sync_copy(cnt_v, cnt_out.at[wid])


@functools.cache
def _sc_reduce_fn():
    return pl.kernel(
        _sc_body,
        out_type=(jax.ShapeDtypeStruct((NW, 2 * MAXP * D), jnp.float32),
                  jax.ShapeDtypeStruct((NW, 16), jnp.float32)),
        mesh=plsc.VectorSubcoreMesh(core_axis_name="c", subcore_axis_name="s",
                                    num_cores=NC, num_subcores=NS),
        compiler_params=pltpu.CompilerParams(needs_layout_passes=False,
                                             skip_device_barrier=True),
        scratch_types=[
            pltpu.VMEM((RPW,), jnp.int32),
            pltpu.VMEM((2 * CH * D,), jnp.float32),
            pltpu.VMEM((MAXP * D,), jnp.float32),
            pltpu.VMEM((MAXP * D,), jnp.float32),
            pltpu.VMEM((16,), jnp.float32),
            pltpu.SemaphoreType.DMA,
            pltpu.SemaphoreType.DMA,
        ],
    )


def _tc_part_body(idc_ref, feats_ref, sum_ref, max_ref, cnt_ref):
    i = pl.program_id(0)

    @pl.when(i == 0)
    def _():
        sum_ref[...] = jnp.zeros_like(sum_ref)
        max_ref[...] = jnp.full_like(max_ref, NEG)
        cnt_ref[...] = jnp.zeros_like(cnt_ref)

    ids1 = idc_ref[...]
    idc = ids1.reshape(16, BT // 16)
    feats = feats_ref[...]
    vr = jnp.minimum(jnp.int32(BT), jnp.int32(NT) - i * BT)
    colio = lax.broadcasted_iota(jnp.int32, (1, BT), 1)
    pid16 = lax.broadcasted_iota(jnp.int32, (16, 1), 0)
    onehot = jnp.where((ids1.reshape(1, BT) == pid16) & (colio < vr),
                       1.0, 0.0)
    sum_ref[...] += jnp.dot(onehot, feats, preferred_element_type=jnp.float32)
    idmin = jnp.min(idc)
    idmax = jnp.max(idc)
    los = [jnp.int32(0)] + [
        jnp.minimum(jnp.sum(jnp.where(idc < p, 1.0, 0.0)).astype(jnp.int32),
                    vr)
        for p in range(1, MAXP)] + [vr]
    m32 = jnp.max(feats.reshape(GB, 32, D), axis=1)
    giota = lax.broadcasted_iota(jnp.int32, (GB, 1), 0)
    wio = lax.broadcasted_iota(jnp.int32, (32, 1), 0)
    for p in range(MAXP):
        lo, hi = los[p], los[p + 1]

        @pl.when((idmin <= p) & (p <= idmax))
        def _():
            gmask = (giota >= (lo + 31) // 32) & (giota < hi // 32)
            cand = jnp.max(jnp.where(gmask, m32, NEG), axis=0, keepdims=True)
            for edge in (lo, hi - 1):
                ws = jnp.minimum(edge // 32, GB - 1) * 32
                win = feats_ref[pl.ds(ws, 32), :]
                wm = ((ws + wio) >= lo) & ((ws + wio) < hi)
                cand = jnp.maximum(
                    cand,
                    jnp.max(jnp.where(wm, win, NEG), axis=0, keepdims=True))
            max_ref[p:p + 1, :] = jnp.maximum(max_ref[p:p + 1, :], cand)
            cnt_ref[p:p + 1, :] += (hi - lo).astype(jnp.float32)


def _tc_partial(idc2, feats):
    return pl.pallas_call(
        _tc_part_body,
        grid=(NBT,),
        in_specs=[
            pl.BlockSpec((BT,), lambda i: (i,)),
            pl.BlockSpec((BT, D), lambda i: (i, 0)),
        ],
        out_specs=[
            pl.BlockSpec((16, D), lambda i: (0, 0)),
            pl.BlockSpec((16, D), lambda i: (0, 0)),
            pl.BlockSpec((16, D), lambda i: (0, 0)),
        ],
        out_shape=[
            jax.ShapeDtypeStruct((16, D), jnp.float32),
            jax.ShapeDtypeStruct((16, D), jnp.float32),
            jax.ShapeDtypeStruct((16, D), jnp.float32),
        ],
        compiler_params=pltpu.CompilerParams(
            dimension_semantics=("arbitrary",)),
    )(idc2, feats)


def _tc_body(part_ref, cnt_ref, tsum_ref, tmax_ref, tcnt_ref,
             np_ref, w_ref, b_ref, o_ref):
    parts = part_ref[...].reshape(NW, 2 * MAXP, D)
    sums = (jnp.sum(parts[:, :MAXP, :], axis=0)
            + tsum_ref[:MAXP, :])
    maxs = jnp.maximum(jnp.max(parts[:, MAXP:, :], axis=0),
                       tmax_ref[:MAXP, :])
    cnt = (jnp.sum(cnt_ref[...], axis=0)[:MAXP]
           + tcnt_ref[:MAXP, 0])
    p_lim = jnp.minimum(np_ref[0, 0], MAXP)
    pidx = lax.broadcasted_iota(jnp.int32, (MAXP, 1), 0)
    valid = (pidx < p_lim) & (cnt[:, None] > 0.0)
    mean = sums / jnp.maximum(cnt[:, None], 1.0)
    mean = jnp.where(valid, mean, 0.0)
    mx = jnp.where(valid, maxs, jnp.float32(-1e9))
    mx = jnp.maximum(mx, 0.0)
    pooled = jnp.concatenate([mean, mx], axis=1)
    out = lax.dot_general(pooled, w_ref[...], (((1,), (1,)), ((), ())),
                          preferred_element_type=jnp.float32)
    o_ref[...] = out + b_ref[...]


def _tc_merge(parts, cnts, tsum, tmax, tcnt, np32, W, b2):
    return pl.pallas_call(
        _tc_body,
        out_shape=jax.ShapeDtypeStruct((MAXP, D), jnp.float32),
        in_specs=[
            pl.BlockSpec(memory_space=pltpu.VMEM),
            pl.BlockSpec(memory_space=pltpu.VMEM),
            pl.BlockSpec(memory_space=pltpu.VMEM),
            pl.BlockSpec(memory_space=pltpu.VMEM),
            pl.BlockSpec(memory_space=pltpu.VMEM),
            pl.BlockSpec(memory_space=pltpu.SMEM),
            pl.BlockSpec(memory_space=pltpu.VMEM),
            pl.BlockSpec(memory_space=pltpu.VMEM),
        ],
    )(parts, cnts, tsum, tmax, tcnt, np32, W, b2)


def kernel(slat_feats, slat_part_ids, num_parts, W, b):
    ids = slat_part_ids.astype(jnp.int32)
    feats_flat = slat_feats.reshape(-1)
    part2d, cnt2d = _sc_reduce_fn()(feats_flat, ids)
    tsum, tmax, tcnt = _tc_partial(ids, slat_feats)
    np32 = jnp.asarray(num_parts, jnp.int32).reshape(1, 1)
    return _tc_merge(part2d, cnt2d, tsum, tmax, tcnt,
                     np32, W, b.reshape(1, D))


# speedup vs baseline: 1.0202x; 1.0202x over previous
import functools

import jax
import jax.numpy as jnp
from jax import lax
from jax.experimental import pallas as pl
from jax.experimental.pallas import tpu as pltpu
from jax.experimental.pallas import tpu_sc as plsc

N = 320000
D = 128
MAXP = 10
NC, NS = 2, 16
NW = NC * NS
CH = 400
SPLIT_M = 9
NSC = 12800 * SPLIT_M
NT = N - NSC
RPW = NSC // NW
NCH = RPW // CH
VPC = CH // 16
KD = D // 16
BT = 8192
NBT = (NT + BT - 1) // BT
GB = BT // 32
NEG = -1e30


def _sc_body(feats_hbm, ids_hbm, part_out, cnt_out,
             ids_v, buf, acc_s, acc_m, cnt_v, sem_a, sem_b):
    cid = lax.axis_index("c")
    sid = lax.axis_index("s")
    wid = cid * NS + sid
    base = NT + wid * RPW
    CHD = CH * D

    pltpu.sync_copy(ids_hbm.at[pl.ds(base, RPW)], ids_v)

    zero = jnp.zeros((16,), jnp.float32)
    neg = jnp.full((16,), NEG, jnp.float32)

    def init_body(r, c):
        acc_s[pl.ds(r * 16, 16)] = zero
        acc_m[pl.ds(r * 16, 16)] = neg
        return c
    lax.fori_loop(0, MAXP * KD, init_body, 0)
    cnt_v[...] = zero
    lanes = lax.iota(jnp.int32, 16)

    def start_copy(g, half, sem):
        pltpu.async_copy(feats_hbm.at[pl.ds((base + g * CH) * D, CHD)],
                         buf.at[pl.ds(half * CHD, CHD)], sem)

    def wait_copy(half, sem):
        pltpu.make_async_copy(feats_hbm.at[pl.ds(0, CHD)],
                              buf.at[pl.ds(half * CHD, CHD)], sem).wait()

    def process(g, half):
        row0 = g * CH
        boff = half * CHD
        id0 = ids_v[pl.ds(row0, 16)][0]
        idl = ids_v[pl.ds(row0 + CH - 16, 16)][15]

        def los_uniform():
            return tuple(jnp.where(id0 >= p, jnp.int32(0), jnp.int32(CH))
                         for p in range(1, MAXP))

        def los_general():
            def cnt_body(v, carry):
                idv = ids_v[pl.ds(row0 + v * 16, 16)]
                return tuple(carry[p - 1]
                             + plsc.all_reduce_population_count(idv < p)
                             for p in range(1, MAXP))
            cvecs = lax.fori_loop(
                0, VPC, cnt_body,
                tuple(jnp.zeros((16,), jnp.int32) for _ in range(MAXP - 1)))
            return tuple(cv[0] for cv in cvecs)

        mids = lax.cond(id0 == idl, los_uniform, los_general)
        los = [jnp.int32(0)] + list(mids) + [jnp.int32(CH)]

        for p in range(MAXP):
            lo, hi = los[p], los[p + 1]

            @pl.when(hi > lo)
            def _():
                def row_body(j, carry):
                    ss, mm = carry
                    off = boff + j * D
                    ss2, mm2 = [], []
                    for k in range(KD):
                        v = buf[pl.ds(off + 16 * k, 16)]
                        ss2.append(ss[k] + v)
                        mm2.append(jnp.maximum(mm[k], v))
                    return tuple(ss2), tuple(mm2)

                ss, mm = lax.fori_loop(
                    lo, hi, row_body,
                    (tuple(zero for _ in range(KD)),
                     tuple(neg for _ in range(KD))))
                for k in range(KD):
                    sl = pl.ds(p * D + 16 * k, 16)
                    acc_s[sl] = acc_s[sl] + ss[k]
                    acc_m[sl] = jnp.maximum(acc_m[sl], mm[k])
                cnt_v[...] = cnt_v[...] + jnp.where(
                    lanes == p, (hi - lo).astype(jnp.float32),
                    jnp.float32(0.0))

    start_copy(0, 0, sem_a)

    def pair_body(gp, c):
        g0 = gp * 2
        start_copy(g0 + 1, 1, sem_b)
        wait_copy(0, sem_a)
        process(g0, 0)
        start_copy(g0 + 2, 0, sem_a)
        wait_copy(1, sem_b)
        process(g0 + 1, 1)
        return c

    lax.fori_loop(0, (NCH - 1) // 2, pair_body, 0)
    wait_copy(0, sem_a)
    process(NCH - 1, 0)

    pltpu.sync_copy(acc_s, part_out.at[wid, pl.ds(0, MAXP * D)])
    pltpu.sync_copy(acc_m, part_out.at[wid, pl.ds(MAXP * D, MAXP * D)])
    pltpu.sync_copy(cnt_v, cnt_out.at[wid])


@functools.cache
def _sc_reduce_fn():
    return pl.kernel(
        _sc_body,
        out_type=(jax.ShapeDtypeStruct((NW, 2 * MAXP * D), jnp.float32),
                  jax.ShapeDtypeStruct((NW, 16), jnp.float32)),
        mesh=plsc.VectorSubcoreMesh(core_axis_name="c", subcore_axis_name="s",
                                    num_cores=NC, num_subcores=NS),
        compiler_params=pltpu.CompilerParams(needs_layout_passes=False,
                                             skip_device_barrier=True),
        scratch_types=[
            pltpu.VMEM((RPW,), jnp.int32),
            pltpu.VMEM((2 * CH * D,), jnp.float32),
            pltpu.VMEM((MAXP * D,), jnp.float32),
            pltpu.VMEM((MAXP * D,), jnp.float32),
            pltpu.VMEM((16,), jnp.float32),
            pltpu.SemaphoreType.DMA,
            pltpu.SemaphoreType.DMA,
        ],
    )


def _tc_part_body(idc_ref, feats_ref, sum_ref, max_ref, cnt_ref):
    i = pl.program_id(0)

    @pl.when(i == 0)
    def _():
        sum_ref[...] = jnp.zeros_like(sum_ref)
        max_ref[...] = jnp.full_like(max_ref, NEG)
        cnt_ref[...] = jnp.zeros_like(cnt_ref)

    ids1 = idc_ref[...]
    idc = ids1.reshape(16, BT // 16)
    feats = feats_ref[...]
    vr = jnp.minimum(jnp.int32(BT), jnp.int32(NT) - i * BT)
    colio = lax.broadcasted_iota(jnp.int32, (1, BT), 1)
    pid16 = lax.broadcasted_iota(jnp.int32, (16, 1), 0)
    onehot = jnp.where((ids1.reshape(1, BT) == pid16) & (colio < vr),
                       1.0, 0.0)
    sum_ref[...] += jnp.dot(onehot, feats, preferred_element_type=jnp.float32)
    idmin = jnp.min(idc)
    idmax = jnp.max(idc)
    los = [jnp.int32(0)] + [
        jnp.minimum(jnp.sum(jnp.where(idc < p, 1.0, 0.0)).astype(jnp.int32),
                    vr)
        for p in range(1, MAXP)] + [vr]
    m32 = jnp.max(feats.reshape(GB, 32, D), axis=1)
    giota = lax.broadcasted_iota(jnp.int32, (GB, 1), 0)
    wio = lax.broadcasted_iota(jnp.int32, (32, 1), 0)
    for p in range(MAXP):
        lo, hi = los[p], los[p + 1]

        @pl.when((idmin <= p) & (p <= idmax))
        def _():
            gmask = (giota >= (lo + 31) // 32) & (giota < hi // 32)
            cand = jnp.max(jnp.where(gmask, m32, NEG), axis=0, keepdims=True)
            for edge in (lo, hi - 1):
                ws = jnp.minimum(edge // 32, GB - 1) * 32
                win = feats_ref[pl.ds(ws, 32), :]
                wm = ((ws + wio) >= lo) & ((ws + wio) < hi)
                cand = jnp.maximum(
                    cand,
                    jnp.max(jnp.where(wm, win, NEG), axis=0, keepdims=True))
            max_ref[p:p + 1, :] = jnp.maximum(max_ref[p:p + 1, :], cand)
            cnt_ref[p:p + 1, :] += (hi - lo).astype(jnp.float32)


def _tc_partial(idc2, feats):
    return pl.pallas_call(
        _tc_part_body,
        grid=(NBT,),
        in_specs=[
            pl.BlockSpec((BT,), lambda i: (i,)),
            pl.BlockSpec((BT, D), lambda i: (i, 0)),
        ],
        out_specs=[
            pl.BlockSpec((16, D), lambda i: (0, 0)),
            pl.BlockSpec((16, D), lambda i: (0, 0)),
            pl.BlockSpec((16, D), lambda i: (0, 0)),
        ],
        out_shape=[
            jax.ShapeDtypeStruct((16, D), jnp.float32),
            jax.ShapeDtypeStruct((16, D), jnp.float32),
            jax.ShapeDtypeStruct((16, D), jnp.float32),
        ],
        compiler_params=pltpu.CompilerParams(
            dimension_semantics=("arbitrary",)),
    )(idc2, feats)


def _tc_body(part_ref, cnt_ref, tsum_ref, tmax_ref, tcnt_ref,
             np_ref, w_ref, b_ref, o_ref):
    parts = part_ref[...].reshape(NW, 2 * MAXP, D)
    sums = (jnp.sum(parts[:, :MAXP, :], axis=0)
            + tsum_ref[:MAXP, :])
    maxs = jnp.maximum(jnp.max(parts[:, MAXP:, :], axis=0),
                       tmax_ref[:MAXP, :])
    cnt = (jnp.sum(cnt_ref[...], axis=0)[:MAXP]
           + tcnt_ref[:MAXP, 0])
    p_lim = jnp.minimum(np_ref[0, 0], MAXP)
    pidx = lax.broadcasted_iota(jnp.int32, (MAXP, 1), 0)
    valid = (pidx < p_lim) & (cnt[:, None] > 0.0)
    mean = sums / jnp.maximum(cnt[:, None], 1.0)
    mean = jnp.where(valid, mean, 0.0)
    mx = jnp.where(valid, maxs, jnp.float32(-1e9))
    mx = jnp.maximum(mx, 0.0)
    pooled = jnp.concatenate([mean, mx], axis=1)
    out = lax.dot_general(pooled, w_ref[...], (((1,), (1,)), ((), ())),
                          preferred_element_type=jnp.float32)
    o_ref[...] = out + b_ref[...]


def _tc_merge(parts, cnts, tsum, tmax, tcnt, np32, W, b2):
    return pl.pallas_call(
        _tc_body,
        out_shape=jax.ShapeDtypeStruct((MAXP, D), jnp.float32),
        in_specs=[
            pl.BlockSpec(memory_space=pltpu.VMEM),
            pl.BlockSpec(memory_space=pltpu.VMEM),
            pl.BlockSpec(memory_space=pltpu.VMEM),
            pl.BlockSpec(memory_space=pltpu.VMEM),
            pl.BlockSpec(memory_space=pltpu.VMEM),
            pl.BlockSpec(memory_space=pltpu.SMEM),
            pl.BlockSpec(memory_space=pltpu.VMEM),
            pl.BlockSpec(memory_space=pltpu.VMEM),
        ],
    )(parts, cnts, tsum, tmax, tcnt, np32, W, b2)


def kernel(slat_feats, slat_part_ids, num_parts, W, b):
    ids = slat_part_ids.astype(jnp.int32)
    feats_flat = slat_feats.reshape(-1)
    part2d, cnt2d = _sc_reduce_fn()(feats_flat, ids)
    tsum, tmax, tcnt = _tc_partial(ids, slat_feats)
    np32 = jnp.asarray(num_parts, jnp.int32).reshape(1, 1)
    return _tc_merge(part2d, cnt2d, tsum, tmax, tcnt,
                     np32, W, b.reshape(1, D))
